# Initial kernel scaffold; baseline (speedup 1.0000x reference)
import functools

import jax
import jax.numpy as jnp
from jax import lax
from jax.experimental import pallas as pl
from jax.experimental.pallas import tpu as pltpu
from jax.experimental.pallas import tpu_sc as plsc

NUM_USERS = 25000
NUM_ITEMS = 25000
N = NUM_USERS + NUM_ITEMS
E = 800000
K = 64
BATCH = 4096

NC = 2
NS = 16
L = 16

HALF = N // NC
HALF_PAD = 25024
TRASH = HALF_PAD
SH_ROWS = 25040
ROWS_PER_TILE_OUT = HALF_PAD // NS
ROWS_PER_TILE_Z = SH_ROWS // NS
ZBLK = 313

EDGES_PER_TILE = E // NS
CHUNK = 80
NCHUNKS = EDGES_PER_TILE // CHUNK

_MESH = plsc.VectorSubcoreMesh(core_axis_name="c", subcore_axis_name="s")


def _lap_body(ego_hbm, src_hbm, dst_hbm, val_hbm, out_hbm,
              src_v, dst_v, val_v, idx_v, rows_v, zero_v, lap_sh, sem):
    c = lax.axis_index("c")
    s = lax.axis_index("s")
    base_row = c * HALF

    def _z(i, _):
        zero_v[pl.ds(i * L, L)] = jnp.zeros((L,), jnp.float32)
        return 0
    lax.fori_loop(0, ZBLK * K // L, _z, 0)
    zrow = s * ROWS_PER_TILE_Z
    z2 = zero_v.reshape(ZBLK, K)
    for r in range(5):
        pltpu.sync_copy(z2, lap_sh.at[pl.ds(zrow + r * ZBLK, ZBLK)])
    plsc.subcore_barrier()

    tile_base = s * EDGES_PER_TILE

    def _chunk(i, _):
        base = pl.multiple_of(tile_base + i * CHUNK, CHUNK)
        pltpu.sync_copy(src_hbm.at[pl.ds(base, CHUNK)], src_v)
        pltpu.sync_copy(dst_hbm.at[pl.ds(base, CHUNK)], dst_v)
        pltpu.sync_copy(val_hbm.at[pl.ds(base, CHUNK)], val_v)
        for g in range(CHUNK // L):
            d = dst_v[pl.ds(g * L, L)]
            loc = d - base_row
            ok = (d >= base_row) & (loc < HALF)
            idx_v[pl.ds(g * L, L)] = jnp.where(ok, loc, TRASH)
        pltpu.async_copy(ego_hbm.at[src_v], rows_v, sem).wait()
        def _scale(e, _):
            v = jnp.full((L,), val_v[e], jnp.float32)
            for j in range(K // L):
                rows_v[e, pl.ds(j * L, L)] = rows_v[e, pl.ds(j * L, L)] * v
            return 0
        lax.fori_loop(0, CHUNK, _scale, 0)
        pltpu.sync_copy(rows_v, lap_sh.at[idx_v], add=True)
        return 0

    lax.fori_loop(0, NCHUNKS, _chunk, 0)
    plsc.subcore_barrier()

    orow = s * ROWS_PER_TILE_OUT
    pltpu.sync_copy(lap_sh.at[pl.ds(orow, ROWS_PER_TILE_OUT)],
                    out_hbm.at[c, pl.ds(orow, ROWS_PER_TILE_OUT)])


_lap_call = pl.kernel(
    _lap_body,
    out_type=jax.ShapeDtypeStruct((NC, HALF_PAD, K), jnp.float32),
    mesh=_MESH,
    scratch_types=[
        pltpu.VMEM((CHUNK,), jnp.int32),
        pltpu.VMEM((CHUNK,), jnp.int32),
        pltpu.VMEM((CHUNK,), jnp.float32),
        pltpu.VMEM((CHUNK,), jnp.int32),
        pltpu.VMEM((CHUNK, K), jnp.float32),
        pltpu.VMEM((ZBLK * K,), jnp.float32),
        pltpu.VMEM_SHARED((SH_ROWS + 8, K), jnp.float32),
        pltpu.SemaphoreType.DMA,
    ],
)


DBLK = 200
DGRID = N // DBLK


def _dense_body(lap_ref, ego_ref, W1_ref, b1_ref, W2_ref, b2_ref,
                ego_out_ref, norm_out_ref):
    lap = lap_ref[0]
    ego = ego_ref[...]
    first = jnp.dot(lap + ego, W1_ref[...],
                    preferred_element_type=jnp.float32) + b1_ref[...]
    second = jnp.dot(ego * lap, W2_ref[...],
                     preferred_element_type=jnp.float32) + b2_ref[...]
    x = first + second
    x = jnp.where(x >= 0, x, 0.2 * x)
    ego_out_ref[...] = x
    ss = jnp.sum(x * x, axis=1, keepdims=True)
    norm_out_ref[...] = x * lax.rsqrt(jnp.maximum(ss, 1e-12))


def _dense_call(lap2, ego, W1, b1, W2, b2):
    per_half = HALF // DBLK
    return pl.pallas_call(
        _dense_body,
        grid=(DGRID,),
        in_specs=[
            pl.BlockSpec((1, DBLK, K), lambda i: (i // per_half, i % per_half, 0)),
            pl.BlockSpec((DBLK, K), lambda i: (i, 0)),
            pl.BlockSpec((K, K), lambda i: (0, 0)),
            pl.BlockSpec((1, K), lambda i: (0, 0)),
            pl.BlockSpec((K, K), lambda i: (0, 0)),
            pl.BlockSpec((1, K), lambda i: (0, 0)),
        ],
        out_specs=[
            pl.BlockSpec((DBLK, K), lambda i: (i, 0)),
            pl.BlockSpec((DBLK, K), lambda i: (i, 0)),
        ],
        out_shape=[
            jax.ShapeDtypeStruct((N, K), jnp.float32),
            jax.ShapeDtypeStruct((N, K), jnp.float32),
        ],
    )(lap2, ego, W1, b1, W2, b2)


BPT = BATCH // (NC * NS)


def _lookup_body(e0_hbm, n1_hbm, n2_hbm, user_hbm, item_hbm,
                 xui_hbm, gu_hbm, gi_hbm,
                 u_idx, i_idx, bufs_u, bufs_i, xui_v, sem):
    c = lax.axis_index("c")
    s = lax.axis_index("s")
    wid = s * NC + c
    base = wid * BPT

    pltpu.sync_copy(user_hbm.at[pl.ds(base, BPT)], u_idx)
    pltpu.sync_copy(item_hbm.at[pl.ds(base, BPT)], i_idx)
    for g in range(BPT // L):
        i_idx[pl.ds(g * L, L)] = i_idx[pl.ds(g * L, L)] + NUM_USERS

    for t, tbl in enumerate((e0_hbm, n1_hbm, n2_hbm)):
        pltpu.async_copy(tbl.at[u_idx], bufs_u.at[t], sem).wait()
        pltpu.async_copy(tbl.at[i_idx], bufs_i.at[t], sem).wait()

    def _dot(e, _):
        acc = jnp.zeros((L,), jnp.float32)
        for t in range(3):
            for j in range(K // L):
                acc = acc + (bufs_u[t, e, pl.ds(j * L, L)] *
                             bufs_i[t, e, pl.ds(j * L, L)])
        xui_v[e] = jnp.sum(acc)
        return 0
    lax.fori_loop(0, BPT, _dot, 0)

    pltpu.sync_copy(xui_v, xui_hbm.at[pl.ds(base, BPT)])
    for t in range(3):
        pltpu.sync_copy(bufs_u.at[t], gu_hbm.at[pl.ds(base, BPT), pl.ds(t * K, K)])
        pltpu.sync_copy(bufs_i.at[t], gi_hbm.at[pl.ds(base, BPT), pl.ds(t * K, K)])


_lookup_call = pl.kernel(
    _lookup_body,
    out_type=(
        jax.ShapeDtypeStruct((BATCH,), jnp.float32),
        jax.ShapeDtypeStruct((BATCH, 3 * K), jnp.float32),
        jax.ShapeDtypeStruct((BATCH, 3 * K), jnp.float32),
    ),
    mesh=_MESH,
    scratch_types=[
        pltpu.VMEM((BPT,), jnp.int32),
        pltpu.VMEM((BPT,), jnp.int32),
        pltpu.VMEM((3, BPT, K), jnp.float32),
        pltpu.VMEM((3, BPT, K), jnp.float32),
        pltpu.VMEM((BPT,), jnp.float32),
        pltpu.SemaphoreType.DMA,
    ],
)


def kernel(gu0, gi0, edge_vals, W1_0, b1_0, W2_0, b2_0, W1_1, b1_1, W2_1, b2_1,
           user, item, edge_index):
    ego0 = jnp.concatenate([gu0, gi0], axis=0)
    src = edge_index[0]
    dst = edge_index[1]

    lap1 = _lap_call(ego0, src, dst, edge_vals)
    ego1, norm1 = _dense_call(lap1, ego0, W1_0, b1_0, W2_0, b2_0)
    lap2 = _lap_call(ego1, src, dst, edge_vals)
    _, norm2 = _dense_call(lap2, ego1, W1_1, b1_1, W2_1, b2_1)

    xui, gamma_u, gamma_i = _lookup_call(ego0, norm1, norm2, user, item)
    return (xui, gamma_u, gamma_i)


# speedup vs baseline: 2.2120x; 2.2120x over previous
import functools

import jax
import jax.numpy as jnp
from jax import lax
from jax.experimental import pallas as pl
from jax.experimental.pallas import tpu as pltpu
from jax.experimental.pallas import tpu_sc as plsc

NUM_USERS = 25000
NUM_ITEMS = 25000
N = NUM_USERS + NUM_ITEMS
E = 800000
K = 64
BATCH = 4096

NC = 2
NS = 16
L = 16

HALF = N // NC
HALF_PAD = 25088
TRASH = HALF_PAD
SH_ROWS = 25600
ROWS_PER_TILE_OUT = HALF_PAD // NS
ROWS_PER_TILE_Z = SH_ROWS // NS
ZBLK = 320

EDGES_PER_TILE = E // NS
CHUNK = 80
NCHUNKS = EDGES_PER_TILE // CHUNK

_MESH = plsc.VectorSubcoreMesh(core_axis_name="c", subcore_axis_name="s")


def _lap_body(ego_hbm, src_hbm, dst_hbm, val_hbm, out_hbm,
              src_v, dst_v, val_v, idx_v, rows_v, zero_v, lap_sh, sem):
    c = lax.axis_index("c")
    s = lax.axis_index("s")
    base_row = c * HALF

    def _z(i, _):
        for j in range(K // L):
            zero_v[i, pl.ds(j * L, L)] = jnp.zeros((L,), jnp.float32)
        return 0
    lax.fori_loop(0, ZBLK, _z, 0)
    zrow = s * ROWS_PER_TILE_Z
    for r in range(5):
        pltpu.sync_copy(zero_v, lap_sh.at[pl.ds(zrow + r * ZBLK, ZBLK)])
    plsc.subcore_barrier()

    tile_base = s * EDGES_PER_TILE

    def _chunk(i, _):
        base = pl.multiple_of(tile_base + i * CHUNK, CHUNK)
        pltpu.sync_copy(src_hbm.at[pl.ds(base, CHUNK)], src_v)
        pltpu.sync_copy(dst_hbm.at[pl.ds(base, CHUNK)], dst_v)
        pltpu.sync_copy(val_hbm.at[pl.ds(base, CHUNK)], val_v)
        for g in range(CHUNK // L):
            d = dst_v[pl.ds(g * L, L)]
            loc = d - base_row
            ok = (d >= base_row) & (loc < HALF)
            idx_v[pl.ds(g * L, L)] = jnp.where(ok, loc, TRASH)
        pltpu.async_copy(ego_hbm.at[src_v], rows_v, sem).wait()
        for g in range(CHUNK // L):
            vals16 = val_v[pl.ds(g * L, L)]
            for l in range(L):
                e = g * L + l
                v = jnp.full((L,), vals16[l], jnp.float32)
                for j in range(K // L):
                    rows_v[e, pl.ds(j * L, L)] = rows_v[e, pl.ds(j * L, L)] * v
        pltpu.sync_copy(rows_v, lap_sh.at[idx_v], add=True)
        return 0

    lax.fori_loop(0, NCHUNKS, _chunk, 0)
    plsc.subcore_barrier()

    orow = s * ROWS_PER_TILE_OUT
    pltpu.sync_copy(lap_sh.at[pl.ds(orow, ROWS_PER_TILE_OUT)],
                    out_hbm.at[c, pl.ds(orow, ROWS_PER_TILE_OUT)])


_lap_call = pl.kernel(
    _lap_body,
    out_type=jax.ShapeDtypeStruct((NC, HALF_PAD, K), jnp.float32),
    mesh=_MESH,
    scratch_types=[
        pltpu.VMEM((CHUNK,), jnp.int32),
        pltpu.VMEM((CHUNK,), jnp.int32),
        pltpu.VMEM((CHUNK,), jnp.float32),
        pltpu.VMEM((CHUNK,), jnp.int32),
        pltpu.VMEM((CHUNK, K), jnp.float32),
        pltpu.VMEM((ZBLK, K), jnp.float32),
        pltpu.VMEM_SHARED((SH_ROWS, K), jnp.float32),
        pltpu.SemaphoreType.DMA,
    ],
    compiler_params=pltpu.CompilerParams(use_tc_tiling_on_sc=False),
)


DBLK = 200
DGRID = N // DBLK


def _dense_body(lap_ref, ego_ref, W1_ref, b1_ref, W2_ref, b2_ref,
                ego_out_ref, norm_out_ref):
    lap = lap_ref[0]
    ego = ego_ref[...]
    first = jnp.dot(lap + ego, W1_ref[...],
                    preferred_element_type=jnp.float32) + b1_ref[...]
    second = jnp.dot(ego * lap, W2_ref[...],
                     preferred_element_type=jnp.float32) + b2_ref[...]
    x = first + second
    x = jnp.where(x >= 0, x, 0.2 * x)
    ego_out_ref[...] = x
    ss = jnp.sum(x * x, axis=1, keepdims=True)
    norm_out_ref[...] = x * lax.rsqrt(jnp.maximum(ss, 1e-12))


def _dense_call(lap2, ego, W1, b1, W2, b2):
    per_half = HALF // DBLK
    return pl.pallas_call(
        _dense_body,
        grid=(DGRID,),
        in_specs=[
            pl.BlockSpec((1, DBLK, K), lambda i: (i // per_half, i % per_half, 0)),
            pl.BlockSpec((DBLK, K), lambda i: (i, 0)),
            pl.BlockSpec((K, K), lambda i: (0, 0)),
            pl.BlockSpec((1, K), lambda i: (0, 0)),
            pl.BlockSpec((K, K), lambda i: (0, 0)),
            pl.BlockSpec((1, K), lambda i: (0, 0)),
        ],
        out_specs=[
            pl.BlockSpec((DBLK, K), lambda i: (i, 0)),
            pl.BlockSpec((DBLK, K), lambda i: (i, 0)),
        ],
        out_shape=[
            jax.ShapeDtypeStruct((N, K), jnp.float32),
            jax.ShapeDtypeStruct((N, K), jnp.float32),
        ],
    )(lap2, ego, W1, b1, W2, b2)


BPT = BATCH // (NC * NS)


def _lookup_body(e0_hbm, n1_hbm, n2_hbm, user_hbm, item_hbm,
                 gu_hbm, gi_hbm,
                 u_idx, i_idx, bufs_u, bufs_i, sem):
    c = lax.axis_index("c")
    s = lax.axis_index("s")
    wid = s * NC + c
    base = wid * BPT

    pltpu.sync_copy(user_hbm.at[pl.ds(base, BPT)], u_idx)
    pltpu.sync_copy(item_hbm.at[pl.ds(base, BPT)], i_idx)
    for g in range(BPT // L):
        i_idx[pl.ds(g * L, L)] = i_idx[pl.ds(g * L, L)] + NUM_USERS

    for t, tbl in enumerate((e0_hbm, n1_hbm, n2_hbm)):
        pltpu.async_copy(tbl.at[u_idx], bufs_u.at[t], sem).wait()
        pltpu.async_copy(tbl.at[i_idx], bufs_i.at[t], sem).wait()

    for t in range(3):
        pltpu.sync_copy(bufs_u.at[t], gu_hbm.at[t, pl.ds(base, BPT)])
        pltpu.sync_copy(bufs_i.at[t], gi_hbm.at[t, pl.ds(base, BPT)])


_lookup_call = pl.kernel(
    _lookup_body,
    out_type=(
        jax.ShapeDtypeStruct((3, BATCH, K), jnp.float32),
        jax.ShapeDtypeStruct((3, BATCH, K), jnp.float32),
    ),
    mesh=_MESH,
    scratch_types=[
        pltpu.VMEM((BPT,), jnp.int32),
        pltpu.VMEM((BPT,), jnp.int32),
        pltpu.VMEM((3, BPT, K), jnp.float32),
        pltpu.VMEM((3, BPT, K), jnp.float32),
        pltpu.SemaphoreType.DMA,
    ],
    compiler_params=pltpu.CompilerParams(use_tc_tiling_on_sc=False),
)


def _xui_body(gu3_ref, gi3_ref, gu_ref, gi_ref, xui_ref):
    gu = jnp.concatenate([gu3_ref[0], gu3_ref[1], gu3_ref[2]], axis=1)
    gi = jnp.concatenate([gi3_ref[0], gi3_ref[1], gi3_ref[2]], axis=1)
    gu_ref[...] = gu
    gi_ref[...] = gi
    xui_ref[...] = jnp.sum(gu * gi, axis=1)


_xui_call = pl.pallas_call(
    _xui_body,
    out_shape=(
        jax.ShapeDtypeStruct((BATCH, 3 * K), jnp.float32),
        jax.ShapeDtypeStruct((BATCH, 3 * K), jnp.float32),
        jax.ShapeDtypeStruct((BATCH,), jnp.float32),
    ),
)


def kernel(gu0, gi0, edge_vals, W1_0, b1_0, W2_0, b2_0, W1_1, b1_1, W2_1, b2_1,
           user, item, edge_index):
    ego0 = jnp.concatenate([gu0, gi0], axis=0)
    src = edge_index[0]
    dst = edge_index[1]

    lap1 = _lap_call(ego0, src, dst, edge_vals)
    ego1, norm1 = _dense_call(lap1, ego0, W1_0, b1_0, W2_0, b2_0)
    lap2 = _lap_call(ego1, src, dst, edge_vals)
    _, norm2 = _dense_call(lap2, ego1, W1_1, b1_1, W2_1, b2_1)

    gu3, gi3 = _lookup_call(ego0, norm1, norm2, user, item)
    gamma_u, gamma_i, xui = _xui_call(gu3, gi3)
    return (xui, gamma_u, gamma_i)


# speedup vs baseline: 2.5852x; 1.1687x over previous
import functools

import jax
import jax.numpy as jnp
from jax import lax
from jax.experimental import pallas as pl
from jax.experimental.pallas import tpu as pltpu
from jax.experimental.pallas import tpu_sc as plsc

NUM_USERS = 25000
NUM_ITEMS = 25000
N = NUM_USERS + NUM_ITEMS
E = 800000
K = 64
BATCH = 4096

NC = 2
NS = 16
L = 16

HALF = N // NC
HALF_PAD = 25088
TRASH = HALF_PAD
SH_ROWS = 25104
ROWS_PER_TILE_OUT = HALF_PAD // NS

EDGES_PER_TILE = E // NS
BLOCK = 80
NBLK = EDGES_PER_TILE // BLOCK
ZROWS = 224

_MESH = plsc.VectorSubcoreMesh(core_axis_name="c", subcore_axis_name="s")


def _lap_body(ego_hbm, src_hbm, dst_hbm, val_hbm, out_hbm,
              src0, src1, dst0, dst1, val0, val1, idx0, idx1,
              rows0, rows1, zero_v, lap_sh,
              lin0, lin1, g0, g1, w0, w1):
    c = lax.axis_index("c")
    s = lax.axis_index("s")
    base_row = c * HALF
    srcv = (src0, src1)
    dstv = (dst0, dst1)
    valv = (val0, val1)
    idx = (idx0, idx1)
    rows = (rows0, rows1)
    lin = (lin0, lin1)
    gsem = (g0, g1)
    wsem = (w0, w1)

    def _z(i, _):
        for j in range(K // L):
            zero_v[i, pl.ds(j * L, L)] = jnp.zeros((L,), jnp.float32)
        return 0
    lax.fori_loop(0, ZROWS, _z, 0)
    zrow = s * ROWS_PER_TILE_OUT
    for r in range(ROWS_PER_TILE_OUT // ZROWS):
        pltpu.sync_copy(zero_v, lap_sh.at[pl.ds(zrow + r * ZROWS, ZROWS)])
    plsc.subcore_barrier()

    tile_base = s * EDGES_PER_TILE

    def _lin_start(b, slot):
        base = tile_base + b * BLOCK
        return (pltpu.async_copy(src_hbm.at[pl.ds(base, BLOCK)], srcv[slot],
                                 lin[slot]),
                pltpu.async_copy(dst_hbm.at[pl.ds(base, BLOCK)], dstv[slot],
                                 lin[slot]),
                pltpu.async_copy(val_hbm.at[pl.ds(base, BLOCK)], valv[slot],
                                 lin[slot]))

    def _gather_start(slot):
        return pltpu.async_copy(ego_hbm.at[srcv[slot]], rows[slot], gsem[slot])

    def _xcompute(slot):
        for g in range(BLOCK // L):
            d = dstv[slot][pl.ds(g * L, L)]
            loc = d - base_row
            ok = (d >= base_row) & (loc < HALF)
            idx[slot][pl.ds(g * L, L)] = jnp.where(ok, loc, TRASH)

    def _scale(slot):
        def _sg(g, _):
            vals16 = valv[slot][pl.ds(g * L, L)]
            for l in range(L):
                e = g * L + l
                v = jnp.full((L,), vals16[l], jnp.float32)
                for j in range(K // L):
                    rows[slot][e, pl.ds(j * L, L)] = (
                        rows[slot][e, pl.ds(j * L, L)] * v)
            return 0
        lax.fori_loop(0, BLOCK // L, _sg, 0)

    def _scatter_start(slot):
        return pltpu.async_copy(rows[slot], lap_sh.at[idx[slot]], wsem[slot],
                                add=True)

    def _pair(i, _):
        ls0 = _lin_start(2 * i, 0)
        ls1 = _lin_start(2 * i + 1, 1)
        for d in ls0:
            d.wait()
        gd0 = _gather_start(0)
        for d in ls1:
            d.wait()
        gd1 = _gather_start(1)
        _xcompute(0)
        _xcompute(1)
        gd0.wait()
        _scale(0)
        wd0 = _scatter_start(0)
        gd1.wait()
        _scale(1)
        wd1 = _scatter_start(1)
        wd0.wait()
        wd1.wait()
        return 0

    lax.fori_loop(0, NBLK // 2, _pair, 0)

    ls = _lin_start(NBLK - 1, 0)
    for d in ls:
        d.wait()
    gd = _gather_start(0)
    _xcompute(0)
    gd.wait()
    _scale(0)
    _scatter_start(0).wait()

    plsc.subcore_barrier()

    orow = s * ROWS_PER_TILE_OUT
    pltpu.sync_copy(lap_sh.at[pl.ds(orow, ROWS_PER_TILE_OUT)],
                    out_hbm.at[c, pl.ds(orow, ROWS_PER_TILE_OUT)])


_lap_call = pl.kernel(
    _lap_body,
    out_type=jax.ShapeDtypeStruct((NC, HALF_PAD, K), jnp.float32),
    mesh=_MESH,
    scratch_types=[
        pltpu.VMEM((BLOCK,), jnp.int32),
        pltpu.VMEM((BLOCK,), jnp.int32),
        pltpu.VMEM((BLOCK,), jnp.int32),
        pltpu.VMEM((BLOCK,), jnp.int32),
        pltpu.VMEM((BLOCK,), jnp.float32),
        pltpu.VMEM((BLOCK,), jnp.float32),
        pltpu.VMEM((BLOCK,), jnp.int32),
        pltpu.VMEM((BLOCK,), jnp.int32),
        pltpu.VMEM((BLOCK, K), jnp.float32),
        pltpu.VMEM((BLOCK, K), jnp.float32),
        pltpu.VMEM((ZROWS, K), jnp.float32),
        pltpu.VMEM_SHARED((SH_ROWS, K), jnp.float32),
        pltpu.SemaphoreType.DMA,
        pltpu.SemaphoreType.DMA,
        pltpu.SemaphoreType.DMA,
        pltpu.SemaphoreType.DMA,
        pltpu.SemaphoreType.DMA,
        pltpu.SemaphoreType.DMA,
    ],
    compiler_params=pltpu.CompilerParams(use_tc_tiling_on_sc=False),
)


DBLK = 200
DGRID = N // DBLK


def _dense_body(lap_ref, ego_ref, W1_ref, b1_ref, W2_ref, b2_ref,
                ego_out_ref, norm_out_ref):
    lap = lap_ref[0]
    ego = ego_ref[...]
    first = jnp.dot(lap + ego, W1_ref[...],
                    preferred_element_type=jnp.float32) + b1_ref[...]
    second = jnp.dot(ego * lap, W2_ref[...],
                     preferred_element_type=jnp.float32) + b2_ref[...]
    x = first + second
    x = jnp.where(x >= 0, x, 0.2 * x)
    ego_out_ref[...] = x
    ss = jnp.sum(x * x, axis=1, keepdims=True)
    norm_out_ref[...] = x * lax.rsqrt(jnp.maximum(ss, 1e-12))


def _dense_call(lap2, ego, W1, b1, W2, b2):
    per_half = HALF // DBLK
    return pl.pallas_call(
        _dense_body,
        grid=(DGRID,),
        in_specs=[
            pl.BlockSpec((1, DBLK, K), lambda i: (i // per_half, i % per_half, 0)),
            pl.BlockSpec((DBLK, K), lambda i: (i, 0)),
            pl.BlockSpec((K, K), lambda i: (0, 0)),
            pl.BlockSpec((1, K), lambda i: (0, 0)),
            pl.BlockSpec((K, K), lambda i: (0, 0)),
            pl.BlockSpec((1, K), lambda i: (0, 0)),
        ],
        out_specs=[
            pl.BlockSpec((DBLK, K), lambda i: (i, 0)),
            pl.BlockSpec((DBLK, K), lambda i: (i, 0)),
        ],
        out_shape=[
            jax.ShapeDtypeStruct((N, K), jnp.float32),
            jax.ShapeDtypeStruct((N, K), jnp.float32),
        ],
    )(lap2, ego, W1, b1, W2, b2)


BPT = BATCH // (NC * NS)


def _lookup_body(e0_hbm, n1_hbm, n2_hbm, user_hbm, item_hbm,
                 gu_hbm, gi_hbm,
                 u_idx, i_idx, bufs_u, bufs_i, sem):
    c = lax.axis_index("c")
    s = lax.axis_index("s")
    wid = s * NC + c
    base = wid * BPT

    pltpu.sync_copy(user_hbm.at[pl.ds(base, BPT)], u_idx)
    pltpu.sync_copy(item_hbm.at[pl.ds(base, BPT)], i_idx)
    for g in range(BPT // L):
        i_idx[pl.ds(g * L, L)] = i_idx[pl.ds(g * L, L)] + NUM_USERS

    for t, tbl in enumerate((e0_hbm, n1_hbm, n2_hbm)):
        pltpu.async_copy(tbl.at[u_idx], bufs_u.at[t], sem).wait()
        pltpu.async_copy(tbl.at[i_idx], bufs_i.at[t], sem).wait()

    for t in range(3):
        pltpu.sync_copy(bufs_u.at[t], gu_hbm.at[t, pl.ds(base, BPT)])
        pltpu.sync_copy(bufs_i.at[t], gi_hbm.at[t, pl.ds(base, BPT)])


_lookup_call = pl.kernel(
    _lookup_body,
    out_type=(
        jax.ShapeDtypeStruct((3, BATCH, K), jnp.float32),
        jax.ShapeDtypeStruct((3, BATCH, K), jnp.float32),
    ),
    mesh=_MESH,
    scratch_types=[
        pltpu.VMEM((BPT,), jnp.int32),
        pltpu.VMEM((BPT,), jnp.int32),
        pltpu.VMEM((3, BPT, K), jnp.float32),
        pltpu.VMEM((3, BPT, K), jnp.float32),
        pltpu.SemaphoreType.DMA,
    ],
    compiler_params=pltpu.CompilerParams(use_tc_tiling_on_sc=False),
)


def _xui_body(gu3_ref, gi3_ref, gu_ref, gi_ref, xui_ref):
    gu = jnp.concatenate([gu3_ref[0], gu3_ref[1], gu3_ref[2]], axis=1)
    gi = jnp.concatenate([gi3_ref[0], gi3_ref[1], gi3_ref[2]], axis=1)
    gu_ref[...] = gu
    gi_ref[...] = gi
    xui_ref[...] = jnp.sum(gu * gi, axis=1)


_xui_call = pl.pallas_call(
    _xui_body,
    out_shape=(
        jax.ShapeDtypeStruct((BATCH, 3 * K), jnp.float32),
        jax.ShapeDtypeStruct((BATCH, 3 * K), jnp.float32),
        jax.ShapeDtypeStruct((BATCH,), jnp.float32),
    ),
)


def kernel(gu0, gi0, edge_vals, W1_0, b1_0, W2_0, b2_0, W1_1, b1_1, W2_1, b2_1,
           user, item, edge_index):
    ego0 = jnp.concatenate([gu0, gi0], axis=0)
    src_ = edge_index[0]
    dst_ = edge_index[1]

    lap1 = _lap_call(ego0, src_, dst_, edge_vals)
    ego1, norm1 = _dense_call(lap1, ego0, W1_0, b1_0, W2_0, b2_0)
    lap2 = _lap_call(ego1, src_, dst_, edge_vals)
    _, norm2 = _dense_call(lap2, ego1, W1_1, b1_1, W2_1, b2_1)

    gu3, gi3 = _lookup_call(ego0, norm1, norm2, user, item)
    gamma_u, gamma_i, xui = _xui_call(gu3, gi3)
    return (xui, gamma_u, gamma_i)


# speedup vs baseline: 2.9619x; 1.1457x over previous
import functools

import jax
import jax.numpy as jnp
from jax import lax
from jax.experimental import pallas as pl
from jax.experimental.pallas import tpu as pltpu
from jax.experimental.pallas import tpu_sc as plsc

NUM_USERS = 25000
NUM_ITEMS = 25000
N = NUM_USERS + NUM_ITEMS
E = 800000
K = 64
BATCH = 4096

NC = 2
NS = 16
L = 16

HALF = N // NC
HALF_PAD = 25088
TRASH = HALF_PAD
SH_ROWS = 25104
ROWS_PER_TILE_OUT = HALF_PAD // NS

EDGES_PER_TILE = E // NS
BLOCK = 80
NBLK = EDGES_PER_TILE // BLOCK
ZROWS = 224

_MESH = plsc.VectorSubcoreMesh(core_axis_name="c", subcore_axis_name="s")


def _lap_body(ego_hbm, src_hbm, dst_hbm, val_hbm, out_hbm,
              src0, src1, dst0, dst1, val0, val1, idx0, idx1,
              rows0, rows1, zero_v, lap_sh,
              lin0, lin1, g0, g1, w0, w1):
    c = lax.axis_index("c")
    s = lax.axis_index("s")
    base_row = c * HALF
    srcv = (src0, src1)
    dstv = (dst0, dst1)
    valv = (val0, val1)
    idx = (idx0, idx1)
    rows = (rows0, rows1)
    lin = (lin0, lin1)
    gsem = (g0, g1)
    wsem = (w0, w1)

    def _z(i, _):
        for j in range(K // L):
            zero_v[i, pl.ds(j * L, L)] = jnp.zeros((L,), jnp.float32)
        return 0
    lax.fori_loop(0, ZROWS, _z, 0)
    zrow = s * ROWS_PER_TILE_OUT
    for r in range(ROWS_PER_TILE_OUT // ZROWS):
        pltpu.sync_copy(zero_v, lap_sh.at[pl.ds(zrow + r * ZROWS, ZROWS)])
    plsc.subcore_barrier()

    tile_base = s * EDGES_PER_TILE

    def _lin_start(b, slot):
        base = tile_base + b * BLOCK
        return (pltpu.async_copy(src_hbm.at[pl.ds(base, BLOCK)], srcv[slot],
                                 lin[slot]),
                pltpu.async_copy(dst_hbm.at[pl.ds(base, BLOCK)], dstv[slot],
                                 lin[slot]),
                pltpu.async_copy(val_hbm.at[pl.ds(base, BLOCK)], valv[slot],
                                 lin[slot]))

    def _gather_start(slot):
        return pltpu.async_copy(ego_hbm.at[srcv[slot]], rows[slot], gsem[slot])

    def _xcompute(slot):
        for g in range(BLOCK // L):
            d = dstv[slot][pl.ds(g * L, L)]
            loc = d - base_row
            ok = (d >= base_row) & (loc < HALF)
            idx[slot][pl.ds(g * L, L)] = jnp.where(ok, loc, TRASH)

    def _scale(slot):
        def _sg(g, _):
            vals16 = valv[slot][pl.ds(g * L, L)]
            for l in range(L):
                e = g * L + l
                v = jnp.full((L,), vals16[l], jnp.float32)
                for j in range(K // L):
                    rows[slot][e, pl.ds(j * L, L)] = (
                        rows[slot][e, pl.ds(j * L, L)] * v)
            return 0
        lax.fori_loop(0, BLOCK // L, _sg, 0)

    def _scatter_start(slot):
        return pltpu.async_copy(rows[slot], lap_sh.at[idx[slot]], wsem[slot],
                                add=True)

    def _lin_wait(slot):
        pltpu.make_async_copy(src_hbm.at[pl.ds(0, BLOCK)], srcv[slot],
                              lin[slot]).wait()
        pltpu.make_async_copy(dst_hbm.at[pl.ds(0, BLOCK)], dstv[slot],
                              lin[slot]).wait()
        pltpu.make_async_copy(val_hbm.at[pl.ds(0, BLOCK)], valv[slot],
                              lin[slot]).wait()

    def _gather_wait(slot):
        pltpu.make_async_copy(ego_hbm.at[srcv[slot]], rows[slot],
                              gsem[slot]).wait()

    def _scatter_wait(slot):
        pltpu.make_async_copy(rows[slot], lap_sh.at[idx[slot]],
                              wsem[slot]).wait()

    def _step(b, slot, first=False, fire_lin=True, fire_g=True):
        other = 1 - slot
        _gather_wait(slot)
        if not first:
            _scatter_wait(other)
        if fire_g:
            _lin_wait(other)
            _gather_start(other)
        _xcompute(slot)
        _scale(slot)
        _scatter_start(slot)
        if fire_lin:
            _lin_start(b + 2, slot)

    _lin_start(0, 0)
    _lin_start(1, 1)
    _lin_wait(0)
    _gather_start(0)
    _step(0, 0, first=True)

    def _pair(i, _):
        b = 2 * i + 1
        _step(b, 1)
        _step(b + 1, 0)
        return 0
    lax.fori_loop(0, (NBLK - 3) // 2, _pair, 0)

    _step(NBLK - 2, 1, fire_lin=False)
    _step(NBLK - 1, 0, fire_lin=False, fire_g=False)
    _scatter_wait(0)

    plsc.subcore_barrier()

    orow = s * ROWS_PER_TILE_OUT
    pltpu.sync_copy(lap_sh.at[pl.ds(orow, ROWS_PER_TILE_OUT)],
                    out_hbm.at[c, pl.ds(orow, ROWS_PER_TILE_OUT)])


_lap_call = pl.kernel(
    _lap_body,
    out_type=jax.ShapeDtypeStruct((NC, HALF_PAD, K), jnp.float32),
    mesh=_MESH,
    scratch_types=[
        pltpu.VMEM((BLOCK,), jnp.int32),
        pltpu.VMEM((BLOCK,), jnp.int32),
        pltpu.VMEM((BLOCK,), jnp.int32),
        pltpu.VMEM((BLOCK,), jnp.int32),
        pltpu.VMEM((BLOCK,), jnp.float32),
        pltpu.VMEM((BLOCK,), jnp.float32),
        pltpu.VMEM((BLOCK,), jnp.int32),
        pltpu.VMEM((BLOCK,), jnp.int32),
        pltpu.VMEM((BLOCK, K), jnp.float32),
        pltpu.VMEM((BLOCK, K), jnp.float32),
        pltpu.VMEM((ZROWS, K), jnp.float32),
        pltpu.VMEM_SHARED((SH_ROWS, K), jnp.float32),
        pltpu.SemaphoreType.DMA,
        pltpu.SemaphoreType.DMA,
        pltpu.SemaphoreType.DMA,
        pltpu.SemaphoreType.DMA,
        pltpu.SemaphoreType.DMA,
        pltpu.SemaphoreType.DMA,
    ],
    compiler_params=pltpu.CompilerParams(use_tc_tiling_on_sc=False),
)


DBLK = 200
DGRID = N // DBLK


def _dense_body(lap_ref, ego_ref, W1_ref, b1_ref, W2_ref, b2_ref,
                ego_out_ref, norm_out_ref):
    lap = lap_ref[0]
    ego = ego_ref[...]
    first = jnp.dot(lap + ego, W1_ref[...],
                    preferred_element_type=jnp.float32) + b1_ref[...]
    second = jnp.dot(ego * lap, W2_ref[...],
                     preferred_element_type=jnp.float32) + b2_ref[...]
    x = first + second
    x = jnp.where(x >= 0, x, 0.2 * x)
    ego_out_ref[...] = x
    ss = jnp.sum(x * x, axis=1, keepdims=True)
    norm_out_ref[...] = x * lax.rsqrt(jnp.maximum(ss, 1e-12))


def _dense_call(lap2, ego, W1, b1, W2, b2):
    per_half = HALF // DBLK
    return pl.pallas_call(
        _dense_body,
        grid=(DGRID,),
        in_specs=[
            pl.BlockSpec((1, DBLK, K), lambda i: (i // per_half, i % per_half, 0)),
            pl.BlockSpec((DBLK, K), lambda i: (i, 0)),
            pl.BlockSpec((K, K), lambda i: (0, 0)),
            pl.BlockSpec((1, K), lambda i: (0, 0)),
            pl.BlockSpec((K, K), lambda i: (0, 0)),
            pl.BlockSpec((1, K), lambda i: (0, 0)),
        ],
        out_specs=[
            pl.BlockSpec((DBLK, K), lambda i: (i, 0)),
            pl.BlockSpec((DBLK, K), lambda i: (i, 0)),
        ],
        out_shape=[
            jax.ShapeDtypeStruct((N, K), jnp.float32),
            jax.ShapeDtypeStruct((N, K), jnp.float32),
        ],
    )(lap2, ego, W1, b1, W2, b2)


BPT = BATCH // (NC * NS)


def _lookup_body(e0_hbm, n1_hbm, n2_hbm, user_hbm, item_hbm,
                 gu_hbm, gi_hbm,
                 u_idx, i_idx, bufs_u, bufs_i, sem):
    c = lax.axis_index("c")
    s = lax.axis_index("s")
    wid = s * NC + c
    base = wid * BPT

    pltpu.sync_copy(user_hbm.at[pl.ds(base, BPT)], u_idx)
    pltpu.sync_copy(item_hbm.at[pl.ds(base, BPT)], i_idx)
    for g in range(BPT // L):
        i_idx[pl.ds(g * L, L)] = i_idx[pl.ds(g * L, L)] + NUM_USERS

    for t, tbl in enumerate((e0_hbm, n1_hbm, n2_hbm)):
        pltpu.async_copy(tbl.at[u_idx], bufs_u.at[t], sem).wait()
        pltpu.async_copy(tbl.at[i_idx], bufs_i.at[t], sem).wait()

    for t in range(3):
        pltpu.sync_copy(bufs_u.at[t], gu_hbm.at[t, pl.ds(base, BPT)])
        pltpu.sync_copy(bufs_i.at[t], gi_hbm.at[t, pl.ds(base, BPT)])


_lookup_call = pl.kernel(
    _lookup_body,
    out_type=(
        jax.ShapeDtypeStruct((3, BATCH, K), jnp.float32),
        jax.ShapeDtypeStruct((3, BATCH, K), jnp.float32),
    ),
    mesh=_MESH,
    scratch_types=[
        pltpu.VMEM((BPT,), jnp.int32),
        pltpu.VMEM((BPT,), jnp.int32),
        pltpu.VMEM((3, BPT, K), jnp.float32),
        pltpu.VMEM((3, BPT, K), jnp.float32),
        pltpu.SemaphoreType.DMA,
    ],
    compiler_params=pltpu.CompilerParams(use_tc_tiling_on_sc=False),
)


def _xui_body(gu3_ref, gi3_ref, gu_ref, gi_ref, xui_ref):
    gu = jnp.concatenate([gu3_ref[0], gu3_ref[1], gu3_ref[2]], axis=1)
    gi = jnp.concatenate([gi3_ref[0], gi3_ref[1], gi3_ref[2]], axis=1)
    gu_ref[...] = gu
    gi_ref[...] = gi
    xui_ref[...] = jnp.sum(gu * gi, axis=1)


_xui_call = pl.pallas_call(
    _xui_body,
    out_shape=(
        jax.ShapeDtypeStruct((BATCH, 3 * K), jnp.float32),
        jax.ShapeDtypeStruct((BATCH, 3 * K), jnp.float32),
        jax.ShapeDtypeStruct((BATCH,), jnp.float32),
    ),
)


def kernel(gu0, gi0, edge_vals, W1_0, b1_0, W2_0, b2_0, W1_1, b1_1, W2_1, b2_1,
           user, item, edge_index):
    ego0 = jnp.concatenate([gu0, gi0], axis=0)
    src_ = edge_index[0]
    dst_ = edge_index[1]

    lap1 = _lap_call(ego0, src_, dst_, edge_vals)
    ego1, norm1 = _dense_call(lap1, ego0, W1_0, b1_0, W2_0, b2_0)
    lap2 = _lap_call(ego1, src_, dst_, edge_vals)
    _, norm2 = _dense_call(lap2, ego1, W1_1, b1_1, W2_1, b2_1)

    gu3, gi3 = _lookup_call(ego0, norm1, norm2, user, item)
    gamma_u, gamma_i, xui = _xui_call(gu3, gi3)
    return (xui, gamma_u, gamma_i)
